# baseline (speedup 1.0000x reference)
import functools
import math

import numpy as np
import jax
import jax.numpy as jnp
from jax import lax
from jax.experimental import pallas as pl
from jax.experimental.pallas import tpu as pltpu
from jax.experimental.pallas import tpu_sc as plsc

_P = 10
_S = 5
_K = 7
_N1 = 15
_N2 = 15
_M = 225
_MP = 240
_C = 8
_H = 80
_W = 80
_HP = 82
_FLAT = _HP * _HP
_FLATP = 6728
_F = 800
_NEG = -1e30


def _build_patch_table():
    q = np.arange(100)
    ph, pw = q // 10, q % 10
    mcol = np.arange(_MP)
    m = np.where(mcol < _M, mcol, 0)
    i, j = m // _N2, m % _N2
    r = 5 * i[None, :] + ph[:, None]
    cc = 5 * j[None, :] + pw[:, None]
    return ((r + 1) * _HP + (cc + 1)).astype(np.int32)


def _build_fold_mats():
    mats = np.zeros((4, _MP, 256), np.float32)
    for t, (di, dj) in enumerate(((0, 0), (0, 1), (1, 0), (1, 1))):
        for i in range(15):
            for j in range(15):
                mats[t, i * _N2 + j, (i + di) * 16 + (j + dj)] = 1.0
    a = np.arange(16)
    cnt = np.where((a == 0) | (a == 15), 1.0, 2.0)
    invcnt = (1.0 / (cnt[:, None] * cnt[None, :])).astype(np.float32)
    return mats, invcnt.reshape(1, 256)


_PATCH_TBL = _build_patch_table().reshape(-1)
_FOLD_P, _FOLD_INVCNT = _build_fold_mats()


def _conv_chain(x_ref, w1_ref, b1_ref, w2_ref, b2_ref, w3_ref, b3_ref,
                mask_ref, out_ref):
    xp = x_ref[0]
    mask = mask_ref[...]

    def conv(inp, w_ref, b_ref, cin):
        wide = jnp.concatenate(
            [jnp.zeros((cin, 128), jnp.float32), inp,
             jnp.zeros((cin, 128), jnp.float32)], axis=1)
        acc = None
        for dy in range(3):
            for dx in range(3):
                s = (dy - 1) * _HP + (dx - 1)
                sl = lax.slice(wide, (0, 128 + s), (cin, 128 + s + _FLATP))
                t = jnp.dot(w_ref[dy * 3 + dx], sl,
                            preferred_element_type=jnp.float32)
                acc = t if acc is None else acc + t
        return acc + b_ref[...]

    y1 = jnp.maximum(conv(xp, w1_ref, b1_ref, 8), 0.0) * mask
    y2 = jnp.maximum(conv(y1, w2_ref, b2_ref, 64), 0.0) * mask
    out_ref[0] = conv(y2, w3_ref, b3_ref, 64)


def _run_convs(x_flat, w1r, b1r, w2r, b2r, w3r, b3r, maskr):
    return pl.pallas_call(
        _conv_chain,
        grid=(2,),
        in_specs=[
            pl.BlockSpec((1, _C, _FLATP), lambda b: (b, 0, 0)),
            pl.BlockSpec((9, 64, 8), lambda b: (0, 0, 0)),
            pl.BlockSpec((64, 1), lambda b: (0, 0)),
            pl.BlockSpec((9, 64, 64), lambda b: (0, 0, 0)),
            pl.BlockSpec((64, 1), lambda b: (0, 0)),
            pl.BlockSpec((9, 8, 64), lambda b: (0, 0, 0)),
            pl.BlockSpec((8, 1), lambda b: (0, 0)),
            pl.BlockSpec((1, _FLATP), lambda b: (0, 0)),
        ],
        out_specs=pl.BlockSpec((1, _C, _FLATP), lambda b: (b, 0, 0)),
        out_shape=jax.ShapeDtypeStruct((2, _C, _FLATP), jnp.float32),
    )(x_flat, w1r, b1r, w2r, b2r, w3r, b3r, maskr)


@functools.lru_cache(maxsize=None)
def _sc_patchify():
    mesh = plsc.VectorSubcoreMesh(core_axis_name="c", subcore_axis_name="s")

    @functools.partial(
        pl.kernel,
        out_type=jax.ShapeDtypeStruct((2, 2, 8, 13 * _C * _MP), jnp.float32),
        mesh=mesh,
        scratch_types=[
            pltpu.VMEM((_C * _FLATP,), jnp.float32),
            pltpu.VMEM((100 * _MP,), jnp.int32),
            pltpu.VMEM((13 * _C * _MP,), jnp.float32),
        ],
        compiler_params=pltpu.CompilerParams(needs_layout_passes=False),
    )
    def _patchify(src_hbm, tbl_hbm, out_hbm, img_v, tbl_v, out_v):
        wid = lax.axis_index("s") * 2 + lax.axis_index("c")
        which = wid // 16
        rem = wid % 16
        b = rem // 8
        g = rem % 8
        pltpu.sync_copy(tbl_hbm, tbl_v)
        pltpu.sync_copy(src_hbm.at[which, b], img_v)

        def row(t, carry):
            q = jnp.minimum(13 * g + t, 99)
            for c in range(_C):
                for mb in range(_MP // 16):
                    idx = tbl_v[pl.ds(q * _MP + mb * 16, 16)] + c * _FLATP
                    out_v[pl.ds((t * _C + c) * _MP + mb * 16, 16)] = (
                        plsc.load_gather(img_v, [idx]))
            return carry

        lax.fori_loop(0, 13, row, 0)
        pltpu.sync_copy(out_v, out_hbm.at[which, b, g])

    return _patchify


def _log1p(y):
    u = 1.0 + y
    d = u - 1.0
    return jnp.where(d == 0.0, y, jnp.log(u) * (y / d))


def _expm1(x):
    u = jnp.exp(x)
    d = u - 1.0
    return jnp.where(d == 0.0, x, d * (x / jnp.log(u)))


def _log1mexp(x, guard=1e-7):
    t = x < math.log(0.5)
    x1 = jnp.where(t, x, -1.0)
    x2 = jnp.where(t, -1.0, x)
    y1 = _log1p(-jnp.exp(x1))
    y2 = jnp.log(-_expm1(x2) + guard)
    return jnp.where(t, y1, y2)


def _knn_body(xfT_ref, xefT_ref, lt_ref, pm_ref, ic_ref, out_ref):
    xfT = lax.slice(xfT_ref[0, 0], (0, 0), (_F, _MP))
    xefT = lax.slice(xefT_ref[0, 0], (0, 0), (_F, _MP))
    temp = jnp.exp(lt_ref[0, 0])

    xe2 = xefT * xefT
    sq_row = jnp.sum(xe2, axis=0, keepdims=True)
    sq_col = lax.dot_general(xe2, jnp.ones((_F, 1), jnp.float32),
                             (((0,), (0,)), ((), ())),
                             precision=lax.Precision.HIGHEST,
                             preferred_element_type=jnp.float32)
    gram = lax.dot_general(xefT, xefT, (((0,), (0,)), ((), ())),
                           preferred_element_type=jnp.float32)

    dfull = (sq_col + sq_row) - 2.0 * gram
    logits = (-dfull) / temp
    ri = lax.broadcasted_iota(jnp.int32, (_MP, _MP), 0)
    ci = lax.broadcasted_iota(jnp.int32, (_MP, _MP), 1)
    kill = jnp.logical_or(ri == ci, ci >= _M)
    logits = jnp.where(kill, _NEG, logits)

    zs = []
    for k in range(_K):
        mx = jnp.max(logits, axis=1, keepdims=True)
        shifted = logits - mx
        sm = jnp.sum(jnp.exp(shifted), axis=1, keepdims=True)
        w = shifted - jnp.log(sm)
        wk = jnp.exp(w)
        zs.append(lax.dot_general(xfT, wk, (((1,), (1,)), ((), ())),
                                  preferred_element_type=jnp.float32))
        if k < _K - 1:
            logits = logits + _log1mexp(w)

    z_all = jnp.concatenate(zs, axis=0)
    acc = None
    for t, (di, dj) in enumerate(((0, 0), (0, 1), (1, 0), (1, 1))):
        pieces = []
        for k in range(_K):
            for ph in range(5 * di, 5 * di + 5):
                r0 = k * _F + ph * 80 + 40 * dj
                pieces.append(lax.slice(z_all, (r0, 0), (r0 + 40, _MP)))
        g = jnp.concatenate(pieces, axis=0)
        y = lax.dot_general(g, pm_ref[t], (((1,), (0,)), ((), ())),
                            preferred_element_type=jnp.float32)
        acc = y if acc is None else acc + y
    out_ref[0] = acc * ic_ref[...]


def _run_knn(xfT, log_temp2d, pmats, invcnt):
    return pl.pallas_call(
        _knn_body,
        grid=(2,),
        in_specs=[
            pl.BlockSpec((1, 1, 832, _MP), lambda b: (0, b, 0, 0)),
            pl.BlockSpec((1, 1, 832, _MP), lambda b: (1, b, 0, 0)),
            pl.BlockSpec((1, 1), lambda b: (0, 0)),
            pl.BlockSpec((4, _MP, 256), lambda b: (0, 0, 0)),
            pl.BlockSpec((1, 256), lambda b: (0, 0)),
        ],
        out_specs=pl.BlockSpec((1, _K * 200, 256), lambda b: (b, 0, 0)),
        out_shape=jax.ShapeDtypeStruct((2, _K * 200, 256), jnp.float32),
    )(xfT, xfT, log_temp2d, pmats, invcnt)


def kernel(x, W1, b1, W2, b2, W3, b3, log_temp):
    x = x.astype(jnp.float32)
    xpad = jnp.pad(x, ((0, 0), (0, 0), (1, 1), (1, 1)))
    x_flat = jnp.pad(xpad.reshape(2, _C, _FLAT), ((0, 0), (0, 0), (0, 4)))

    w1r = W1.transpose(2, 3, 0, 1).reshape(9, 64, 8).astype(jnp.float32)
    w2r = W2.transpose(2, 3, 0, 1).reshape(9, 64, 64).astype(jnp.float32)
    w3r = W3.transpose(2, 3, 0, 1).reshape(9, 8, 64).astype(jnp.float32)
    b1r = b1.reshape(64, 1).astype(jnp.float32)
    b2r = b2.reshape(64, 1).astype(jnp.float32)
    b3r = b3.reshape(8, 1).astype(jnp.float32)

    interior = np.zeros((_HP, _HP), np.float32)
    interior[1:-1, 1:-1] = 1.0
    maskr = jnp.asarray(
        np.pad(interior.reshape(-1), (0, 4)).reshape(1, _FLATP))

    xe_flat = _run_convs(x_flat, w1r, b1r, w2r, b2r, w3r, b3r, maskr)

    src = jnp.stack([x_flat, xe_flat]).reshape(2, 2, _C * _FLATP)
    xfT = _sc_patchify()(src, jnp.asarray(_PATCH_TBL))
    xfT = xfT.reshape(2, 2, 104 * _C, _MP)

    planes = _run_knn(xfT, log_temp.reshape(1, 1).astype(jnp.float32),
                      jnp.asarray(_FOLD_P),
                      jnp.asarray(_FOLD_INVCNT))

    zimg = (planes.reshape(2, _K, 5, 5, _C, 16, 16)
            .transpose(0, 1, 4, 5, 2, 6, 3)
            .reshape(2, _K * _C, _H, _W))
    return jnp.concatenate([x, zimg], axis=1)


# speedup vs baseline: 1.0624x; 1.0624x over previous
import functools
import math

import numpy as np
import jax
import jax.numpy as jnp
from jax import lax
from jax.experimental import pallas as pl
from jax.experimental.pallas import tpu as pltpu
from jax.experimental.pallas import tpu_sc as plsc

_P = 10
_S = 5
_K = 7
_N1 = 15
_N2 = 15
_M = 225
_MP = 240
_C = 8
_H = 80
_W = 80
_HP = 82
_FLAT = _HP * _HP
_FLATP = 6728
_F = 800
_NEG = -1e30


def _build_patch_table():
    q = np.arange(100)
    ph, pw = q // 10, q % 10
    mcol = np.arange(_MP)
    m = np.where(mcol < _M, mcol, 0)
    i, j = m // _N2, m % _N2
    r = 5 * i[None, :] + ph[:, None]
    cc = 5 * j[None, :] + pw[:, None]
    return ((r + 1) * _HP + (cc + 1)).astype(np.int32)


def _build_fold_mats():
    mats = np.zeros((4, _MP, 256), np.float32)
    for t, (di, dj) in enumerate(((0, 0), (0, 1), (1, 0), (1, 1))):
        for i in range(15):
            for j in range(15):
                mats[t, i * _N2 + j, (i + di) * 16 + (j + dj)] = 1.0
    a = np.arange(16)
    cnt = np.where((a == 0) | (a == 15), 1.0, 2.0)
    invcnt = (1.0 / (cnt[:, None] * cnt[None, :])).astype(np.float32)
    return mats, invcnt.reshape(1, 256)


_PATCH_TBL = _build_patch_table().reshape(-1)
_FOLD_P, _FOLD_INVCNT = _build_fold_mats()


def _conv_chain(x_ref, w1_ref, b1_ref, w2_ref, b2_ref, w3_ref, b3_ref,
                mask_ref, out_ref):
    xp = x_ref[0]
    mask = mask_ref[...]

    def conv(inp, w_ref, b_ref, cin):
        wide = jnp.concatenate(
            [jnp.zeros((cin, 128), jnp.float32), inp,
             jnp.zeros((cin, 128), jnp.float32)], axis=1)
        acc = None
        for dy in range(3):
            for dx in range(3):
                s = (dy - 1) * _HP + (dx - 1)
                sl = lax.slice(wide, (0, 128 + s), (cin, 128 + s + _FLATP))
                t = jnp.dot(w_ref[dy * 3 + dx], sl,
                            preferred_element_type=jnp.float32)
                acc = t if acc is None else acc + t
        return acc + b_ref[...]

    y1 = jnp.maximum(conv(xp, w1_ref, b1_ref, 8), 0.0) * mask
    y2 = jnp.maximum(conv(y1, w2_ref, b2_ref, 64), 0.0) * mask
    out_ref[0] = conv(y2, w3_ref, b3_ref, 64)


def _run_convs(x_flat, w1r, b1r, w2r, b2r, w3r, b3r, maskr):
    return pl.pallas_call(
        _conv_chain,
        grid=(2,),
        in_specs=[
            pl.BlockSpec((1, _C, _FLATP), lambda b: (b, 0, 0)),
            pl.BlockSpec((9, 64, 8), lambda b: (0, 0, 0)),
            pl.BlockSpec((64, 1), lambda b: (0, 0)),
            pl.BlockSpec((9, 64, 64), lambda b: (0, 0, 0)),
            pl.BlockSpec((64, 1), lambda b: (0, 0)),
            pl.BlockSpec((9, 8, 64), lambda b: (0, 0, 0)),
            pl.BlockSpec((8, 1), lambda b: (0, 0)),
            pl.BlockSpec((1, _FLATP), lambda b: (0, 0)),
        ],
        out_specs=pl.BlockSpec((1, _C, _FLATP), lambda b: (b, 0, 0)),
        out_shape=jax.ShapeDtypeStruct((2, _C, _FLATP), jnp.float32),
    )(x_flat, w1r, b1r, w2r, b2r, w3r, b3r, maskr)


@functools.lru_cache(maxsize=None)
def _sc_patchify():
    mesh = plsc.VectorSubcoreMesh(core_axis_name="c", subcore_axis_name="s")

    @functools.partial(
        pl.kernel,
        out_type=jax.ShapeDtypeStruct((2, 2, 8, 13 * _C * _MP), jnp.float32),
        mesh=mesh,
        scratch_types=[
            pltpu.VMEM((_C * _FLATP,), jnp.float32),
            pltpu.VMEM((100 * _MP,), jnp.int32),
            pltpu.VMEM((13 * _C * _MP,), jnp.float32),
        ],
        compiler_params=pltpu.CompilerParams(needs_layout_passes=False),
    )
    def _patchify(src_hbm, tbl_hbm, out_hbm, img_v, tbl_v, out_v):
        wid = lax.axis_index("s") * 2 + lax.axis_index("c")
        which = wid // 16
        rem = wid % 16
        b = rem // 8
        g = rem % 8
        pltpu.sync_copy(tbl_hbm, tbl_v)
        pltpu.sync_copy(src_hbm.at[which, b], img_v)

        def row(t, carry):
            q = jnp.minimum(13 * g + t, 99)
            for c in range(_C):
                for mb in range(_MP // 16):
                    idx = tbl_v[pl.ds(q * _MP + mb * 16, 16)] + c * _FLATP
                    out_v[pl.ds((t * _C + c) * _MP + mb * 16, 16)] = (
                        plsc.load_gather(img_v, [idx]))
            return carry

        lax.fori_loop(0, 13, row, 0)
        pltpu.sync_copy(out_v, out_hbm.at[which, b, g])

    return _patchify


def _log1p(y):
    u = 1.0 + y
    d = u - 1.0
    return jnp.where(d == 0.0, y, jnp.log(u) * (y / d))


def _expm1(x):
    u = jnp.exp(x)
    d = u - 1.0
    return jnp.where(d == 0.0, x, d * (x / jnp.log(u)))


def _log1mexp(x, guard=1e-7):
    t = x < math.log(0.5)
    x1 = jnp.where(t, x, -1.0)
    x2 = jnp.where(t, -1.0, x)
    y1 = _log1p(-jnp.exp(x1))
    y2 = jnp.log(-_expm1(x2) + guard)
    return jnp.where(t, y1, y2)


def _knn_body(xfT_ref, xefT_ref, lt_ref, pm_ref, ic_ref, out_ref):
    xfT = lax.slice(xfT_ref[0, 0], (0, 0), (_F, _MP))
    xefT = lax.slice(xefT_ref[0, 0], (0, 0), (_F, _MP))
    temp = jnp.exp(lt_ref[0, 0])

    xe2 = xefT * xefT
    sq_row = jnp.sum(xe2, axis=0, keepdims=True)
    sq_col = lax.dot_general(xe2, jnp.ones((_F, 1), jnp.float32),
                             (((0,), (0,)), ((), ())),
                             precision=lax.Precision.HIGHEST,
                             preferred_element_type=jnp.float32)
    gram = lax.dot_general(xefT, xefT, (((0,), (0,)), ((), ())),
                           preferred_element_type=jnp.float32)

    dfull = (sq_col + sq_row) - 2.0 * gram
    logits = (-dfull) / temp
    ri = lax.broadcasted_iota(jnp.int32, (_MP, _MP), 0)
    ci = lax.broadcasted_iota(jnp.int32, (_MP, _MP), 1)
    kill = jnp.logical_or(ri == ci, ci >= _M)
    logits = jnp.where(kill, _NEG, logits)

    zs = []
    for k in range(_K):
        mx = jnp.max(logits, axis=1, keepdims=True)
        shifted = logits - mx
        sm = jnp.sum(jnp.exp(shifted), axis=1, keepdims=True)
        w = shifted - jnp.log(sm)
        wk = jnp.exp(w)
        zs.append(lax.dot_general(xfT, wk, (((1,), (1,)), ((), ())),
                                  preferred_element_type=jnp.float32))
        if k < _K - 1:
            logits = logits + _log1mexp(w)

    z_all = jnp.concatenate(zs, axis=0)
    acc = None
    for t, (di, dj) in enumerate(((0, 0), (0, 1), (1, 0), (1, 1))):
        pieces = []
        for k in range(_K):
            for ph in range(5 * di, 5 * di + 5):
                r0 = k * _F + ph * 80 + 40 * dj
                pieces.append(lax.slice(z_all, (r0, 0), (r0 + 40, _MP)))
        g = jnp.concatenate(pieces, axis=0)
        y = lax.dot_general(g, pm_ref[t], (((1,), (0,)), ((), ())),
                            preferred_element_type=jnp.float32)
        acc = y if acc is None else acc + y
    out_ref[0] = acc * ic_ref[...]


def _run_knn(xfT, log_temp2d, pmats, invcnt):
    return pl.pallas_call(
        _knn_body,
        grid=(2,),
        in_specs=[
            pl.BlockSpec((1, 1, 832, _MP), lambda b: (0, b, 0, 0)),
            pl.BlockSpec((1, 1, 832, _MP), lambda b: (1, b, 0, 0)),
            pl.BlockSpec((1, 1), lambda b: (0, 0)),
            pl.BlockSpec((4, _MP, 256), lambda b: (0, 0, 0)),
            pl.BlockSpec((1, 256), lambda b: (0, 0)),
        ],
        out_specs=pl.BlockSpec((1, _K * 200, 256), lambda b: (b, 0, 0)),
        out_shape=jax.ShapeDtypeStruct((2, _K * 200, 256), jnp.float32),
    )(xfT, xfT, log_temp2d, pmats, invcnt)


def kernel(x, W1, b1, W2, b2, W3, b3, log_temp):
    x = x.astype(jnp.float32)
    xpad = jnp.pad(x, ((0, 0), (0, 0), (1, 1), (1, 1)))
    x_flat = jnp.pad(xpad.reshape(2, _C, _FLAT), ((0, 0), (0, 0), (0, 4)))

    w1r = W1.transpose(2, 3, 0, 1).reshape(9, 64, 8).astype(jnp.float32)
    w2r = W2.transpose(2, 3, 0, 1).reshape(9, 64, 64).astype(jnp.float32)
    w3r = W3.transpose(2, 3, 0, 1).reshape(9, 8, 64).astype(jnp.float32)
    b1r = b1.reshape(64, 1).astype(jnp.float32)
    b2r = b2.reshape(64, 1).astype(jnp.float32)
    b3r = b3.reshape(8, 1).astype(jnp.float32)

    interior = np.zeros((_HP, _HP), np.float32)
    interior[1:-1, 1:-1] = 1.0
    maskr = jnp.asarray(
        np.pad(interior.reshape(-1), (0, 4)).reshape(1, _FLATP))

    xe_flat = _run_convs(x_flat, w1r, b1r, w2r, b2r, w3r, b3r, maskr)

    src = jnp.stack([x_flat, xe_flat]).reshape(2, 2, _C * _FLATP)
    xfT = _sc_patchify()(src, jnp.asarray(_PATCH_TBL))
    xfT = xfT.reshape(2, 2, 104 * _C, _MP)

    planes = _run_knn(xfT, log_temp.reshape(1, 1).astype(jnp.float32),
                      jnp.asarray(_FOLD_P),
                      jnp.asarray(_FOLD_INVCNT))

    t0 = planes.reshape(2, _K, 25, _C, 256).transpose(0, 1, 3, 2, 4)
    t0 = lax.optimization_barrier(t0)
    t1 = t0.reshape(2, _K, _C, 5, 5, 256).transpose(0, 1, 2, 3, 5, 4)
    t1 = lax.optimization_barrier(t1)
    t2 = (t1.reshape(2, _K, _C, 5, 16, 80)
          .transpose(0, 1, 2, 4, 3, 5))
    zimg = t2.reshape(2, _K * _C, _H, _W)
    return jnp.concatenate([x, zimg], axis=1)


# speedup vs baseline: 1.5261x; 1.4364x over previous
import functools
import math

import numpy as np
import jax
import jax.numpy as jnp
from jax import lax
from jax.experimental import pallas as pl
from jax.experimental.pallas import tpu as pltpu
from jax.experimental.pallas import tpu_sc as plsc

_P = 10
_S = 5
_K = 7
_N1 = 15
_N2 = 15
_M = 225
_MP = 240
_C = 8
_H = 80
_W = 80
_HP = 82
_FLAT = _HP * _HP
_FLATP = 6728
_F = 800
_NEG = -1e30


def _build_patch_table():
    q = np.arange(100)
    ph, pw = q // 10, q % 10
    mcol = np.arange(_MP)
    m = np.where(mcol < _M, mcol, 0)
    i, j = m // _N2, m % _N2
    r = 5 * i[None, :] + ph[:, None]
    cc = 5 * j[None, :] + pw[:, None]
    return ((r + 1) * _HP + (cc + 1)).astype(np.int32)


def _build_fold_mats():
    mats = np.zeros((4, _MP, 256), np.float32)
    for t, (di, dj) in enumerate(((0, 0), (0, 1), (1, 0), (1, 1))):
        for i in range(15):
            for j in range(15):
                mats[t, i * _N2 + j, (i + di) * 16 + (j + dj)] = 1.0
    a = np.arange(16)
    cnt = np.where((a == 0) | (a == 15), 1.0, 2.0)
    invcnt = (1.0 / (cnt[:, None] * cnt[None, :])).astype(np.float32)
    return mats, invcnt.reshape(1, 256)


def _build_deint_table():
    p = np.arange(_H * _W)
    r, cc = p // _W, p % _W
    u, a = r % 5, r // 5
    v, bb = cc % 5, cc // 5
    return (((u * 5 + v) * _C) * 256 + a * 16 + bb).astype(np.int32)


_PATCH_TBL = _build_patch_table().reshape(-1)
_FOLD_P, _FOLD_INVCNT = _build_fold_mats()
_DEINT_TBL = _build_deint_table()


def _conv_chain(x_ref, w1_ref, b1_ref, w2_ref, b2_ref, w3_ref, b3_ref,
                mask_ref, out_ref):
    xp = x_ref[0]
    mask = mask_ref[...]

    def conv(inp, w_ref, b_ref, cin):
        wide = jnp.concatenate(
            [jnp.zeros((cin, 128), jnp.float32), inp,
             jnp.zeros((cin, 128), jnp.float32)], axis=1)
        acc = None
        for dy in range(3):
            for dx in range(3):
                s = (dy - 1) * _HP + (dx - 1)
                sl = lax.slice(wide, (0, 128 + s), (cin, 128 + s + _FLATP))
                t = jnp.dot(w_ref[dy * 3 + dx], sl,
                            preferred_element_type=jnp.float32)
                acc = t if acc is None else acc + t
        return acc + b_ref[...]

    y1 = jnp.maximum(conv(xp, w1_ref, b1_ref, 8), 0.0) * mask
    y2 = jnp.maximum(conv(y1, w2_ref, b2_ref, 64), 0.0) * mask
    out_ref[0] = conv(y2, w3_ref, b3_ref, 64)


def _run_convs(x_flat, w1r, b1r, w2r, b2r, w3r, b3r, maskr):
    return pl.pallas_call(
        _conv_chain,
        grid=(2,),
        in_specs=[
            pl.BlockSpec((1, _C, _FLATP), lambda b: (b, 0, 0)),
            pl.BlockSpec((9, 64, 8), lambda b: (0, 0, 0)),
            pl.BlockSpec((64, 1), lambda b: (0, 0)),
            pl.BlockSpec((9, 64, 64), lambda b: (0, 0, 0)),
            pl.BlockSpec((64, 1), lambda b: (0, 0)),
            pl.BlockSpec((9, 8, 64), lambda b: (0, 0, 0)),
            pl.BlockSpec((8, 1), lambda b: (0, 0)),
            pl.BlockSpec((1, _FLATP), lambda b: (0, 0)),
        ],
        out_specs=pl.BlockSpec((1, _C, _FLATP), lambda b: (b, 0, 0)),
        out_shape=jax.ShapeDtypeStruct((2, _C, _FLATP), jnp.float32),
    )(x_flat, w1r, b1r, w2r, b2r, w3r, b3r, maskr)


@functools.lru_cache(maxsize=None)
def _sc_patchify():
    mesh = plsc.VectorSubcoreMesh(core_axis_name="c", subcore_axis_name="s")

    @functools.partial(
        pl.kernel,
        out_type=jax.ShapeDtypeStruct((2, 2, 8, 13 * _C * _MP), jnp.float32),
        mesh=mesh,
        scratch_types=[
            pltpu.VMEM((_C * _FLATP,), jnp.float32),
            pltpu.VMEM((100 * _MP,), jnp.int32),
            pltpu.VMEM((13 * _C * _MP,), jnp.float32),
        ],
        compiler_params=pltpu.CompilerParams(needs_layout_passes=False),
    )
    def _patchify(src_hbm, tbl_hbm, out_hbm, img_v, tbl_v, out_v):
        wid = lax.axis_index("s") * 2 + lax.axis_index("c")
        which = wid // 16
        rem = wid % 16
        b = rem // 8
        g = rem % 8
        pltpu.sync_copy(tbl_hbm, tbl_v)
        pltpu.sync_copy(src_hbm.at[which, b], img_v)

        def row(t, carry):
            q = jnp.minimum(13 * g + t, 99)
            for c in range(_C):
                for mb in range(_MP // 16):
                    idx = tbl_v[pl.ds(q * _MP + mb * 16, 16)] + c * _FLATP
                    out_v[pl.ds((t * _C + c) * _MP + mb * 16, 16)] = (
                        plsc.load_gather(img_v, [idx]))
            return carry

        lax.fori_loop(0, 13, row, 0)
        pltpu.sync_copy(out_v, out_hbm.at[which, b, g])

    return _patchify


def _log1p(y):
    u = 1.0 + y
    d = u - 1.0
    return jnp.where(d == 0.0, y, jnp.log(u) * (y / d))


def _expm1(x):
    u = jnp.exp(x)
    d = u - 1.0
    return jnp.where(d == 0.0, x, d * (x / jnp.log(u)))


def _log1mexp(x, guard=1e-7):
    t = x < math.log(0.5)
    x1 = jnp.where(t, x, -1.0)
    x2 = jnp.where(t, -1.0, x)
    y1 = _log1p(-jnp.exp(x1))
    y2 = jnp.log(-_expm1(x2) + guard)
    return jnp.where(t, y1, y2)


def _knn_body(xfT_ref, xefT_ref, lt_ref, pm_ref, ic_ref, out_ref):
    xfT = lax.slice(xfT_ref[0, 0], (0, 0), (_F, _MP))
    xefT = lax.slice(xefT_ref[0, 0], (0, 0), (_F, _MP))
    temp = jnp.exp(lt_ref[0, 0])

    xe2 = xefT * xefT
    sq_row = jnp.sum(xe2, axis=0, keepdims=True)
    sq_col = lax.dot_general(xe2, jnp.ones((_F, 1), jnp.float32),
                             (((0,), (0,)), ((), ())),
                             precision=lax.Precision.HIGHEST,
                             preferred_element_type=jnp.float32)
    gram = lax.dot_general(xefT, xefT, (((0,), (0,)), ((), ())),
                           preferred_element_type=jnp.float32)

    dfull = (sq_col + sq_row) - 2.0 * gram
    logits = (-dfull) / temp
    ri = lax.broadcasted_iota(jnp.int32, (_MP, _MP), 0)
    ci = lax.broadcasted_iota(jnp.int32, (_MP, _MP), 1)
    kill = jnp.logical_or(ri == ci, ci >= _M)
    logits = jnp.where(kill, _NEG, logits)

    zs = []
    for k in range(_K):
        mx = jnp.max(logits, axis=1, keepdims=True)
        shifted = logits - mx
        sm = jnp.sum(jnp.exp(shifted), axis=1, keepdims=True)
        w = shifted - jnp.log(sm)
        wk = jnp.exp(w)
        zs.append(lax.dot_general(xfT, wk, (((1,), (1,)), ((), ())),
                                  preferred_element_type=jnp.float32))
        if k < _K - 1:
            logits = logits + _log1mexp(w)

    z_all = jnp.concatenate(zs, axis=0)
    acc = None
    for t, (di, dj) in enumerate(((0, 0), (0, 1), (1, 0), (1, 1))):
        pieces = []
        for k in range(_K):
            for ph in range(5 * di, 5 * di + 5):
                r0 = k * _F + ph * 80 + 40 * dj
                pieces.append(lax.slice(z_all, (r0, 0), (r0 + 40, _MP)))
        g = jnp.concatenate(pieces, axis=0)
        y = lax.dot_general(g, pm_ref[t], (((1,), (0,)), ((), ())),
                            preferred_element_type=jnp.float32)
        acc = y if acc is None else acc + y
    out_ref[0] = acc * ic_ref[...]


def _run_knn(xfT, log_temp2d, pmats, invcnt):
    return pl.pallas_call(
        _knn_body,
        grid=(2,),
        in_specs=[
            pl.BlockSpec((1, 1, 832, _MP), lambda b: (0, b, 0, 0)),
            pl.BlockSpec((1, 1, 832, _MP), lambda b: (1, b, 0, 0)),
            pl.BlockSpec((1, 1), lambda b: (0, 0)),
            pl.BlockSpec((4, _MP, 256), lambda b: (0, 0, 0)),
            pl.BlockSpec((1, 256), lambda b: (0, 0)),
        ],
        out_specs=pl.BlockSpec((1, _K * 200, 256), lambda b: (b, 0, 0)),
        out_shape=jax.ShapeDtypeStruct((2, _K * 200, 256), jnp.float32),
    )(xfT, xfT, log_temp2d, pmats, invcnt)


@functools.lru_cache(maxsize=None)
def _sc_deinterleave():
    mesh = plsc.VectorSubcoreMesh(core_axis_name="c", subcore_axis_name="s")

    @functools.partial(
        pl.kernel,
        out_type=jax.ShapeDtypeStruct((2, _K, 2, 4 * _H * _W), jnp.float32),
        mesh=mesh,
        scratch_types=[
            pltpu.VMEM((200 * 256,), jnp.float32),
            pltpu.VMEM((_H * _W,), jnp.int32),
            pltpu.VMEM((4 * _H * _W,), jnp.float32),
        ],
        compiler_params=pltpu.CompilerParams(needs_layout_passes=False),
    )
    def _deint(planes_hbm, tbl_hbm, out_hbm, blk_v, tbl_v, out_v):
        wid = lax.axis_index("s") * 2 + lax.axis_index("c")

        @pl.when(wid < 2 * _K * 2)
        def _():
            b = wid // (_K * 2)
            rem = wid % (_K * 2)
            k = rem // 2
            half = rem % 2
            pltpu.sync_copy(tbl_hbm, tbl_v)
            pltpu.sync_copy(planes_hbm.at[b, k], blk_v)

            for c4 in range(4):
                coff = (4 * half + c4) * 256

                def chunk(t, carry, c4=c4, coff=coff):
                    sl = pl.ds(t * 16, 16)
                    idx = tbl_v[sl] + coff
                    out_v[pl.ds(c4 * (_H * _W) + t * 16, 16)] = (
                        plsc.load_gather(blk_v, [idx]))
                    return carry

                lax.fori_loop(0, (_H * _W) // 16, chunk, 0)
            pltpu.sync_copy(out_v, out_hbm.at[b, k, half])

    return _deint


def kernel(x, W1, b1, W2, b2, W3, b3, log_temp):
    x = x.astype(jnp.float32)
    xpad = jnp.pad(x, ((0, 0), (0, 0), (1, 1), (1, 1)))
    x_flat = jnp.pad(xpad.reshape(2, _C, _FLAT), ((0, 0), (0, 0), (0, 4)))

    w1r = W1.transpose(2, 3, 0, 1).reshape(9, 64, 8).astype(jnp.float32)
    w2r = W2.transpose(2, 3, 0, 1).reshape(9, 64, 64).astype(jnp.float32)
    w3r = W3.transpose(2, 3, 0, 1).reshape(9, 8, 64).astype(jnp.float32)
    b1r = b1.reshape(64, 1).astype(jnp.float32)
    b2r = b2.reshape(64, 1).astype(jnp.float32)
    b3r = b3.reshape(8, 1).astype(jnp.float32)

    interior = np.zeros((_HP, _HP), np.float32)
    interior[1:-1, 1:-1] = 1.0
    maskr = jnp.asarray(
        np.pad(interior.reshape(-1), (0, 4)).reshape(1, _FLATP))

    xe_flat = _run_convs(x_flat, w1r, b1r, w2r, b2r, w3r, b3r, maskr)

    src = jnp.stack([x_flat, xe_flat]).reshape(2, 2, _C * _FLATP)
    xfT = _sc_patchify()(src, jnp.asarray(_PATCH_TBL))
    xfT = xfT.reshape(2, 2, 104 * _C, _MP)

    planes = _run_knn(xfT, log_temp.reshape(1, 1).astype(jnp.float32),
                      jnp.asarray(_FOLD_P),
                      jnp.asarray(_FOLD_INVCNT))

    zimg = _sc_deinterleave()(
        planes.reshape(2, _K, 200 * 256),
        jnp.asarray(_DEINT_TBL)).reshape(2, _K * _C, _H, _W)
    return jnp.concatenate([x, zimg], axis=1)
